# baseline (speedup 1.0000x reference)
import functools

import jax
import jax.numpy as jnp
from jax import lax
from jax.experimental import pallas as pl
from jax.experimental.pallas import tpu as pltpu
from jax.experimental.pallas import tpu_sc as plsc

B, C, H, W = 4, 64, 24, 24
HW = H * W
K = 512
_N_ELEM = B * C * HW

_NW = 32
_QPW = B * HW // _NW
_QPAD = 80


def _vq_tc_kernel(x_ref, emb_ref, idx_ref, tail_ref, loss_ref):
    emb = emb_ref[...]
    en = jnp.sum(emb * emb, axis=1, keepdims=True)
    rowids = lax.broadcasted_iota(jnp.int32, (K, HW), 0)
    big = jnp.int32(K)

    total = jnp.zeros((1, 1), jnp.float32)
    for b in range(B):
        xb = x_ref[b]

        scores = lax.dot_general(
            emb, xb, (((1,), (0,)), ((), ())),
            preferred_element_type=jnp.float32,
            precision=lax.Precision.HIGHEST,
        )
        d = en - 2.0 * scores

        dmin1 = jnp.min(d, axis=0, keepdims=True)
        i1 = jnp.min(jnp.where(d == dmin1, rowids, big), axis=0, keepdims=True)
        oh1 = (rowids == i1).astype(jnp.float32)
        e1 = lax.dot_general(
            emb, oh1, (((0,), (0,)), ((), ())),
            preferred_element_type=jnp.float32,
            precision=lax.Precision.HIGHEST,
        )

        dm = jnp.where(rowids == i1, jnp.float32(jnp.inf), d)
        dmin2 = jnp.min(dm, axis=0, keepdims=True)
        i2 = jnp.min(jnp.where(dm == dmin2, rowids, big), axis=0, keepdims=True)
        oh2 = (rowids == i2).astype(jnp.float32)
        e2 = lax.dot_general(
            emb, oh2, (((0,), (0,)), ((), ())),
            preferred_element_type=jnp.float32,
            precision=lax.Precision.HIGHEST,
        )

        d1 = jnp.sum((xb - e1) ** 2, axis=0, keepdims=True)
        d2 = jnp.sum((xb - e2) ** 2, axis=0, keepdims=True)
        win2 = (d2 < d1) | ((d2 == d1) & (i2 < i1))

        idx_ref[b] = jnp.where(win2, i2, i1)

        tail_ref[b] = jnp.where(win2[:, _SC_Q:], e2[:, _SC_Q:], e1[:, _SC_Q:])

        dwin = jnp.where(win2, d2, d1)
        total = total + jnp.sum(dwin, axis=1, keepdims=True)

    loss_ref[...] = total * jnp.float32(2.0 / _N_ELEM)


_QFULL = 128
_SC_Q = 512
_CHUNKS = _SC_Q // _QFULL


@functools.partial(
    pl.kernel,
    out_type=jax.ShapeDtypeStruct((B, C, _SC_Q), jnp.float32),
    mesh=plsc.VectorSubcoreMesh(core_axis_name="c", subcore_axis_name="s"),
    scratch_types=[
        pltpu.VMEM((HW,), jnp.int32),
        pltpu.VMEM((_QFULL, 2 * C), jnp.float32),
        pltpu.VMEM((C, _QFULL), jnp.float32),
        pltpu.SemaphoreType.DMA,
    ],
    compiler_params=pltpu.CompilerParams(needs_layout_passes=False),
)
def _sc_gather(emb_hbm, idx_hbm, out_hbm, idx_v, rows_v, out_v, sem):
    wid = lax.axis_index("s") * 2 + lax.axis_index("c")
    b = wid // _CHUNKS
    chunk = wid % _CHUNKS
    hw0 = chunk * _QFULL

    @pl.when(wid < B * _CHUNKS)
    def _work():
        pltpu.sync_copy(idx_hbm.at[b, 0], idx_v)
        pltpu.async_copy(emb_hbm.at[idx_v.at[pl.ds(hw0, _QFULL)]],
                         rows_v, sem).wait()
        for j in range(_QFULL // 16):
            qvec = jax.lax.iota(jnp.int32, 16) + (j * 16)
            for c in range(C):
                col = jnp.zeros((16,), jnp.int32) + c
                out_v[c, pl.ds(j * 16, 16)] = plsc.load_gather(
                    rows_v, [qvec, col])
        pltpu.sync_copy(out_v, out_hbm.at[b, :, pl.ds(hw0, _QFULL)])


@jax.jit
def kernel(x, embeddings):
    x3 = x.reshape(B, C, HW)
    idx3, tail, loss = pl.pallas_call(
        _vq_tc_kernel,
        out_shape=[
            jax.ShapeDtypeStruct((B, 1, HW), jnp.int32),
            jax.ShapeDtypeStruct((B, C, HW - _SC_Q), jnp.float32),
            jax.ShapeDtypeStruct((1, 1), jnp.float32),
        ],
    )(x3, embeddings)
    emb_pad = jnp.pad(embeddings, ((0, 0), (0, C)))
    zq_main = _sc_gather(emb_pad, idx3)
    zq3 = jnp.concatenate([zq_main, tail], axis=2)
    return zq3.reshape(B, C, H, W), loss[0, 0]


# speedup vs baseline: 1.0932x; 1.0932x over previous
import functools

import jax
import jax.numpy as jnp
from jax import lax
from jax.experimental import pallas as pl
from jax.experimental.pallas import tpu as pltpu
from jax.experimental.pallas import tpu_sc as plsc

B, C, H, W = 4, 64, 24, 24
HW = H * W
K = 512
_N_ELEM = B * C * HW

_NW = 32
_QPW = B * HW // _NW
_QPAD = 80


def _vq_tc_kernel(x_ref, emb_ref, idx_ref, tail_ref, loss_ref):
    emb = emb_ref[...]
    en = jnp.sum(emb * emb, axis=1, keepdims=True)
    rowids = lax.broadcasted_iota(jnp.int32, (K, HW), 0)
    big = jnp.int32(K)

    total = jnp.zeros((1, 1), jnp.float32)
    for b in range(B):
        xb = x_ref[b]

        scores = lax.dot_general(
            emb, xb, (((1,), (0,)), ((), ())),
            preferred_element_type=jnp.float32,
            precision=lax.Precision.HIGHEST,
        )
        d = en - 2.0 * scores

        dmin1 = jnp.min(d, axis=0, keepdims=True)
        i1 = jnp.min(jnp.where(d == dmin1, rowids, big), axis=0, keepdims=True)
        oh1 = (rowids == i1).astype(jnp.float32)
        e1 = lax.dot_general(
            emb, oh1, (((0,), (0,)), ((), ())),
            preferred_element_type=jnp.float32,
            precision=lax.Precision.HIGHEST,
        )

        dm = jnp.where(rowids == i1, jnp.float32(jnp.inf), d)
        dmin2 = jnp.min(dm, axis=0, keepdims=True)
        i2 = jnp.min(jnp.where(dm == dmin2, rowids, big), axis=0, keepdims=True)
        oh2 = (rowids == i2).astype(jnp.float32)
        e2 = lax.dot_general(
            emb, oh2, (((0,), (0,)), ((), ())),
            preferred_element_type=jnp.float32,
            precision=lax.Precision.HIGHEST,
        )

        d1 = jnp.sum((xb - e1) ** 2, axis=0, keepdims=True)
        d2 = jnp.sum((xb - e2) ** 2, axis=0, keepdims=True)
        win2 = (d2 < d1) | ((d2 == d1) & (i2 < i1))

        idx_ref[b] = jnp.where(win2, i2, i1)

        tail_ref[b] = jnp.where(win2[:, _SC_Q:], e2[:, _SC_Q:], e1[:, _SC_Q:])

        dwin = jnp.where(win2, d2, d1)
        total = total + jnp.sum(dwin, axis=1, keepdims=True)

    loss_ref[...] = total * jnp.float32(2.0 / _N_ELEM)


_QFULL = 128
_SC_Q = 512
_CHUNKS = _SC_Q // _QFULL


@functools.partial(
    pl.kernel,
    out_type=jax.ShapeDtypeStruct((B, C, _SC_Q), jnp.float32),
    mesh=plsc.VectorSubcoreMesh(core_axis_name="c", subcore_axis_name="s"),
    scratch_types=[
        pltpu.VMEM((HW,), jnp.int32),
        pltpu.VMEM((_QFULL, 2 * C), jnp.float32),
        pltpu.VMEM((C, _QFULL), jnp.float32),
        pltpu.SemaphoreType.DMA,
    ],
    compiler_params=pltpu.CompilerParams(needs_layout_passes=False),
)
def _sc_gather(emb_hbm, idx_hbm, out_hbm, idx_v, rows_v, out_v, sem):
    wid = lax.axis_index("s") * 2 + lax.axis_index("c")
    b = wid // _CHUNKS
    chunk = wid % _CHUNKS
    hw0 = chunk * _QFULL

    @pl.when(wid < B * _CHUNKS)
    def _work():
        pltpu.sync_copy(idx_hbm.at[b, 0], idx_v)
        pltpu.async_copy(emb_hbm.at[idx_v.at[pl.ds(hw0, _QFULL)]],
                         rows_v, sem).wait()
        lanes = jax.lax.iota(jnp.int32, 16)

        @plsc.parallel_loop(0, (_QFULL // 16) * (C // 16), unroll=2)
        def _transpose(it):
            j = it // (C // 16)
            c0 = (it % (C // 16)) * 16
            qvec = lanes + j * 16
            for cc in range(16):
                col = jnp.zeros((16,), jnp.int32) + (c0 + cc)
                out_v[c0 + cc, pl.ds(j * 16, 16)] = plsc.load_gather(
                    rows_v, [qvec, col])
        pltpu.sync_copy(out_v, out_hbm.at[b, :, pl.ds(hw0, _QFULL)])


@jax.jit
def kernel(x, embeddings):
    x3 = x.reshape(B, C, HW)
    idx3, tail, loss = pl.pallas_call(
        _vq_tc_kernel,
        out_shape=[
            jax.ShapeDtypeStruct((B, 1, HW), jnp.int32),
            jax.ShapeDtypeStruct((B, C, HW - _SC_Q), jnp.float32),
            jax.ShapeDtypeStruct((1, 1), jnp.float32),
        ],
    )(x3, embeddings)
    emb_pad = jnp.pad(embeddings, ((0, 0), (0, C)))
    zq_main = _sc_gather(emb_pad, idx3)
    zq3 = jnp.concatenate([zq_main, tail], axis=2)
    return zq3.reshape(B, C, H, W), loss[0, 0]


# speedup vs baseline: 1.9413x; 1.7758x over previous
import functools

import jax
import jax.numpy as jnp
from jax import lax
from jax.experimental import pallas as pl

B, C, H, W = 4, 64, 24, 24
HW = H * W
K = 512
_N_ELEM = B * C * HW


def _vq_tc_kernel(x_ref, emb_ref, zq_ref, loss_ref):
    emb = emb_ref[...]
    en = jnp.sum(emb * emb, axis=1, keepdims=True)
    rowids = lax.broadcasted_iota(jnp.int32, (K, HW), 0)
    big = jnp.int32(K)

    total = jnp.zeros((1, 1), jnp.float32)
    for b in range(B):
        xb = x_ref[b]

        scores = lax.dot_general(
            emb, xb, (((1,), (0,)), ((), ())),
            preferred_element_type=jnp.float32,
            precision=lax.Precision.HIGHEST,
        )
        d = en - 2.0 * scores

        dmin1 = jnp.min(d, axis=0, keepdims=True)
        i1 = jnp.min(jnp.where(d == dmin1, rowids, big), axis=0, keepdims=True)
        oh1 = (rowids == i1).astype(jnp.float32)
        e1 = lax.dot_general(
            emb, oh1, (((0,), (0,)), ((), ())),
            preferred_element_type=jnp.float32,
            precision=lax.Precision.HIGHEST,
        )

        dm = jnp.where(rowids == i1, jnp.float32(jnp.inf), d)
        dmin2 = jnp.min(dm, axis=0, keepdims=True)
        i2 = jnp.min(jnp.where(dm == dmin2, rowids, big), axis=0, keepdims=True)
        oh2 = (rowids == i2).astype(jnp.float32)
        e2 = lax.dot_general(
            emb, oh2, (((0,), (0,)), ((), ())),
            preferred_element_type=jnp.float32,
            precision=lax.Precision.HIGHEST,
        )

        d1 = jnp.sum((xb - e1) ** 2, axis=0, keepdims=True)
        d2 = jnp.sum((xb - e2) ** 2, axis=0, keepdims=True)
        win2 = (d2 < d1) | ((d2 == d1) & (i2 < i1))

        zqb = jnp.where(win2, e2, e1)
        for h in range(H):
            zq_ref[b, :, h, :] = zqb[:, h * W:(h + 1) * W]

        dwin = jnp.where(win2, d2, d1)
        total = total + jnp.sum(dwin, axis=1, keepdims=True)

    loss_ref[...] = total * jnp.float32(2.0 / _N_ELEM)


@jax.jit
def kernel(x, embeddings):
    x3 = x.reshape(B, C, HW)
    zq4, loss = pl.pallas_call(
        _vq_tc_kernel,
        out_shape=[
            jax.ShapeDtypeStruct((B, C, H, W), jnp.float32),
            jax.ShapeDtypeStruct((1, 1), jnp.float32),
        ],
    )(x3, embeddings)
    return zq4, loss[0, 0]


# speedup vs baseline: 2.5792x; 1.3286x over previous
import functools

import jax
import jax.numpy as jnp
from jax import lax
from jax.experimental import pallas as pl

B, C, H, W = 4, 64, 24, 24
HW = H * W
K = 512
_N_ELEM = B * C * HW


def _pick(emb_parts, onehot):
    acc = None
    for part in emb_parts:
        t = lax.dot_general(
            part, onehot, (((0,), (0,)), ((), ())),
            preferred_element_type=jnp.float32,
        )
        acc = t if acc is None else acc + t
    return acc


def _vq_tc_kernel(x_ref, emb_ref, zq_ref, loss_ref):
    emb = emb_ref[...]
    en = jnp.sum(emb * emb, axis=1, keepdims=True)
    rowids = lax.broadcasted_iota(jnp.int32, (K, HW), 0)
    big = jnp.int32(K)

    e_hi = emb.astype(jnp.bfloat16)
    r1 = emb - e_hi.astype(jnp.float32)
    e_mid = r1.astype(jnp.bfloat16)
    e_lo = (r1 - e_mid.astype(jnp.float32)).astype(jnp.bfloat16)
    emb_parts = (e_hi, e_mid, e_lo)

    total = jnp.zeros((1, 1), jnp.float32)
    for b in range(B):
        xb = x_ref[b]

        scores = lax.dot_general(
            emb, xb, (((1,), (0,)), ((), ())),
            preferred_element_type=jnp.float32,
            precision=lax.Precision.HIGHEST,
        )
        d = en - 2.0 * scores

        dmin1 = jnp.min(d, axis=0, keepdims=True)
        i1 = jnp.min(jnp.where(d == dmin1, rowids, big), axis=0, keepdims=True)
        oh1 = (rowids == i1).astype(jnp.bfloat16)
        e1 = _pick(emb_parts, oh1)

        dm = jnp.where(rowids == i1, jnp.float32(jnp.inf), d)
        dmin2 = jnp.min(dm, axis=0, keepdims=True)
        i2 = jnp.min(jnp.where(dm == dmin2, rowids, big), axis=0, keepdims=True)
        oh2 = (rowids == i2).astype(jnp.bfloat16)
        e2 = _pick(emb_parts, oh2)

        d1 = jnp.sum((xb - e1) ** 2, axis=0, keepdims=True)
        d2 = jnp.sum((xb - e2) ** 2, axis=0, keepdims=True)
        win2 = (d2 < d1) | ((d2 == d1) & (i2 < i1))

        zq_ref[b] = jnp.where(win2, e2, e1)

        dwin = jnp.where(win2, d2, d1)
        total = total + jnp.sum(dwin, axis=1, keepdims=True)

    loss_ref[...] = total * jnp.float32(2.0 / _N_ELEM)


@jax.jit
def kernel(x, embeddings):
    x3 = x.reshape(B, C, HW)
    zq3, loss = pl.pallas_call(
        _vq_tc_kernel,
        out_shape=[
            jax.ShapeDtypeStruct((B, C, HW), jnp.float32),
            jax.ShapeDtypeStruct((1, 1), jnp.float32),
        ],
    )(x3, embeddings)
    return zq3.reshape(B, C, H, W), loss[0, 0]


# speedup vs baseline: 3.0657x; 1.1886x over previous
import functools

import jax
import jax.numpy as jnp
from jax import lax
from jax.experimental import pallas as pl

B, C, H, W = 4, 64, 24, 24
HW = H * W
K = 512
_N_ELEM = B * C * HW


def _pick(emb_parts, onehot):
    acc = None
    for part in emb_parts:
        t = lax.dot_general(
            part, onehot, (((0,), (0,)), ((), ())),
            preferred_element_type=jnp.float32,
        )
        acc = t if acc is None else acc + t
    return acc


def _vq_tc_kernel(x_ref, emb_ref, zq_ref, loss_ref):
    emb = emb_ref[...]
    en = jnp.sum(emb * emb, axis=1, keepdims=True)
    rowids = lax.broadcasted_iota(jnp.int32, (K, HW), 0)
    big = jnp.int32(K)

    e_hi = emb.astype(jnp.bfloat16)
    r1 = emb - e_hi.astype(jnp.float32)
    e_mid = r1.astype(jnp.bfloat16)
    e_lo = (r1 - e_mid.astype(jnp.float32)).astype(jnp.bfloat16)
    emb_parts = (e_hi, e_mid, e_lo)
    e6 = jnp.concatenate([e_hi, e_hi, e_mid, e_hi, e_lo, e_mid], axis=1)

    total = jnp.zeros((1, 1), jnp.float32)
    for b in range(B):
        xb = x_ref[b]

        x_hi = xb.astype(jnp.bfloat16)
        xr = xb - x_hi.astype(jnp.float32)
        x_mid = xr.astype(jnp.bfloat16)
        x_lo = (xr - x_mid.astype(jnp.float32)).astype(jnp.bfloat16)
        x6 = jnp.concatenate([x_hi, x_mid, x_hi, x_lo, x_hi, x_mid], axis=0)

        scores = lax.dot_general(
            e6, x6, (((1,), (0,)), ((), ())),
            preferred_element_type=jnp.float32,
        )
        d = en - 2.0 * scores

        dmin1 = jnp.min(d, axis=0, keepdims=True)
        i1 = jnp.min(jnp.where(d == dmin1, rowids, big), axis=0, keepdims=True)
        oh1 = (rowids == i1).astype(jnp.bfloat16)
        e1 = _pick(emb_parts, oh1)

        dm = jnp.where(rowids == i1, jnp.float32(jnp.inf), d)
        dmin2 = jnp.min(dm, axis=0, keepdims=True)
        i2 = jnp.min(jnp.where(dm == dmin2, rowids, big), axis=0, keepdims=True)
        oh2 = (rowids == i2).astype(jnp.bfloat16)
        e2 = _pick(emb_parts, oh2)

        d1 = jnp.sum((xb - e1) ** 2, axis=0, keepdims=True)
        d2 = jnp.sum((xb - e2) ** 2, axis=0, keepdims=True)
        win2 = (d2 < d1) | ((d2 == d1) & (i2 < i1))

        zq_ref[b] = jnp.where(win2, e2, e1)

        dwin = jnp.where(win2, d2, d1)
        total = total + jnp.sum(dwin, axis=1, keepdims=True)

    loss_ref[...] = total * jnp.float32(2.0 / _N_ELEM)


@jax.jit
def kernel(x, embeddings):
    x3 = x.reshape(B, C, HW)
    zq3, loss = pl.pallas_call(
        _vq_tc_kernel,
        out_shape=[
            jax.ShapeDtypeStruct((B, C, HW), jnp.float32),
            jax.ShapeDtypeStruct((1, 1), jnp.float32),
        ],
    )(x3, embeddings)
    return zq3.reshape(B, C, H, W), loss[0, 0]
